# Initial kernel scaffold; baseline (speedup 1.0000x reference)
import functools

import jax
import jax.numpy as jnp
from jax import lax
from jax.experimental import pallas as pl
from jax.experimental.pallas import tpu as pltpu
from jax.experimental.pallas import tpu_sc as plsc

HIDDEN = 64
N_CAT = 8
VOCAB = 100000
N_SCAL = 13
BATCH = 16384

NC, NS, LANES = 2, 16, 16
NW = NC * NS
BPW = BATCH // NW
RPW = BPW * N_CAT
CHUNK = 128
NCHUNK = RPW // CHUNK
NVEC = RPW // LANES

X_COLS = N_CAT + N_SCAL


def _sc_gather(x, tab_flat):
    mesh = plsc.VectorSubcoreMesh(core_axis_name="c", subcore_axis_name="s")

    @functools.partial(
        pl.kernel,
        out_type=jax.ShapeDtypeStruct((BATCH * N_CAT, HIDDEN), jnp.float32),
        mesh=mesh,
        scratch_types=[
            pltpu.VMEM((BPW, X_COLS), jnp.float32),
            pltpu.VMEM((NCHUNK, CHUNK), jnp.int32),
            pltpu.VMEM((2, CHUNK, HIDDEN), jnp.float32),
            pltpu.SemaphoreType.DMA,
            pltpu.SemaphoreType.DMA,
            pltpu.SemaphoreType.DMA,
            pltpu.SemaphoreType.DMA,
        ],
    )
    def k(x_hbm, tab_hbm, out_hbm, x_v, idx_v, rows_v, sg0, sg1, so0, so1):
        wid = lax.axis_index("s") * NC + lax.axis_index("c")
        base = wid * BPW
        obase = base * N_CAT

        pltpu.sync_copy(x_hbm.at[pl.ds(base, BPW)], x_v)

        lane = lax.iota(jnp.int32, LANES)

        @pl.loop(0, NVEC)
        def _(v):
            p = v * LANES + lane
            b = p >> 3
            t = p & 7
            val = plsc.load_gather(x_v, [b, t])
            idx = val.astype(jnp.int32) + t * VOCAB
            idx_v[v >> 3, pl.ds((v & 7) * LANES, LANES)] = idx

        sg = (sg0, sg1)
        so = (so0, so1)

        def g_start(g, s):
            pltpu.async_copy(tab_hbm.at[idx_v.at[g]], rows_v.at[s], sg[s])

        def g_wait(s):
            pltpu.make_async_copy(
                tab_hbm.at[idx_v.at[0]], rows_v.at[s], sg[s]).wait()

        def o_start(g, s):
            pltpu.async_copy(
                rows_v.at[s], out_hbm.at[pl.ds(obase + g * CHUNK, CHUNK)], so[s])

        def o_wait(s):
            pltpu.make_async_copy(
                rows_v.at[s], out_hbm.at[pl.ds(obase, CHUNK)], so[s]).wait()

        g_start(0, 0)
        g_start(1, 1)

        @pl.loop(0, NCHUNK - 2, step=2)
        def _(g):
            for s in (0, 1):
                gg = g + s
                g_wait(s)
                o_start(gg, s)
                o_wait(s)
                g_start(gg + 2, s)

        for s, gg in ((0, NCHUNK - 2), (1, NCHUNK - 1)):
            g_wait(s)
            o_start(gg, s)
        for s in (0, 1):
            o_wait(s)

    return k(x, tab_flat)


def _tc_project(g2, xs, wd1, wl, bl2, wd2, bd2):
    BM = 2048

    def body(g_ref, xs_ref, wd1_ref, wl_ref, bl_ref, wd2_ref, bd_ref, o_ref):
        scal = (
            jnp.dot(xs_ref[...], wl_ref[...], preferred_element_type=jnp.float32)
            + bl_ref[...]
        )
        acc = jnp.dot(g_ref[...], wd1_ref[...], preferred_element_type=jnp.float32)
        acc = acc + jnp.dot(scal, wd2_ref[...], preferred_element_type=jnp.float32)
        o_ref[...] = acc + bd_ref[...]

    d1 = N_CAT * HIDDEN
    return pl.pallas_call(
        body,
        grid=(BATCH // BM,),
        in_specs=[
            pl.BlockSpec((BM, d1), lambda i: (i, 0)),
            pl.BlockSpec((BM, N_SCAL), lambda i: (i, 0)),
            pl.BlockSpec((d1, HIDDEN), lambda i: (0, 0)),
            pl.BlockSpec((N_SCAL, HIDDEN), lambda i: (0, 0)),
            pl.BlockSpec((1, HIDDEN), lambda i: (0, 0)),
            pl.BlockSpec((HIDDEN, HIDDEN), lambda i: (0, 0)),
            pl.BlockSpec((1, HIDDEN), lambda i: (0, 0)),
        ],
        out_specs=pl.BlockSpec((BM, HIDDEN), lambda i: (i, 0)),
        out_shape=jax.ShapeDtypeStruct((BATCH, HIDDEN), jnp.float32),
    )(g2, xs, wd1, wl, bl2, wd2, bd2)


def kernel(x, tables, Wl, bl, Wd, bd):
    tab_flat = tables.reshape(N_CAT * VOCAB, HIDDEN)
    gathered = _sc_gather(x, tab_flat)
    g2 = gathered.reshape(BATCH, N_CAT * HIDDEN)
    xs = x[:, N_CAT:]
    wd1 = Wd[: N_CAT * HIDDEN]
    wd2 = Wd[N_CAT * HIDDEN :]
    return _tc_project(
        g2, xs, wd1, Wl, bl.reshape(1, HIDDEN), wd2, bd.reshape(1, HIDDEN)
    )


# speedup vs baseline: 1.5686x; 1.5686x over previous
import functools

import jax
import jax.numpy as jnp
from jax import lax
from jax.experimental import pallas as pl
from jax.experimental.pallas import tpu as pltpu
from jax.experimental.pallas import tpu_sc as plsc

HIDDEN = 64
N_CAT = 8
VOCAB = 100000
N_SCAL = 13
BATCH = 16384

NC, NS, LANES = 2, 16, 16
NW = NC * NS
BPW = BATCH // NW
RPW = BPW * N_CAT
CHUNK = 128
NCHUNK = RPW // CHUNK
NVEC = RPW // LANES

X_COLS = N_CAT + N_SCAL


def _sc_gather(x, tab_flat):
    mesh = plsc.VectorSubcoreMesh(core_axis_name="c", subcore_axis_name="s")

    @functools.partial(
        pl.kernel,
        out_type=jax.ShapeDtypeStruct((BATCH * N_CAT, HIDDEN), jnp.float32),
        mesh=mesh,
        scratch_types=[
            pltpu.VMEM((BPW * X_COLS,), jnp.float32),
            pltpu.VMEM((NCHUNK, CHUNK), jnp.int32),
            pltpu.VMEM((2, CHUNK, HIDDEN), jnp.float32),
            pltpu.SemaphoreType.DMA,
            pltpu.SemaphoreType.DMA,
            pltpu.SemaphoreType.DMA,
            pltpu.SemaphoreType.DMA,
        ],
        compiler_params=pltpu.CompilerParams(
            needs_layout_passes=False, use_tc_tiling_on_sc=False
        ),
    )
    def k(x_hbm, tab_hbm, out_hbm, x_v, idx_v, rows_v, sg0, sg1, so0, so1):
        wid = lax.axis_index("s") * NC + lax.axis_index("c")
        base = wid * BPW
        obase = base * N_CAT

        pltpu.sync_copy(x_hbm.at[pl.ds(base * X_COLS, BPW * X_COLS)], x_v)

        lane = lax.iota(jnp.int32, LANES)

        @pl.loop(0, NVEC)
        def _(v):
            p = v * LANES + lane
            b = p >> 3
            t = p & 7
            val = plsc.load_gather(x_v, [b * X_COLS + t])
            idx = val.astype(jnp.int32) + t * VOCAB
            idx_v[v >> 3, pl.ds((v & 7) * LANES, LANES)] = idx

        sg = (sg0, sg1)
        so = (so0, so1)

        def g_start(g, s):
            pltpu.async_copy(tab_hbm.at[idx_v.at[g]], rows_v.at[s], sg[s])

        def g_wait(s):
            pltpu.make_async_copy(
                tab_hbm.at[idx_v.at[0]], rows_v.at[s], sg[s]).wait()

        def o_start(g, s):
            pltpu.async_copy(
                rows_v.at[s], out_hbm.at[pl.ds(obase + g * CHUNK, CHUNK)], so[s])

        def o_wait(s):
            pltpu.make_async_copy(
                rows_v.at[s], out_hbm.at[pl.ds(obase, CHUNK)], so[s]).wait()

        g_start(0, 0)
        g_start(1, 1)

        @pl.loop(0, NCHUNK - 2, step=2)
        def _(g):
            for s in (0, 1):
                gg = g + s
                g_wait(s)
                o_start(gg, s)
                o_wait(s)
                g_start(gg + 2, s)

        for s, gg in ((0, NCHUNK - 2), (1, NCHUNK - 1)):
            g_wait(s)
            o_start(gg, s)
        for s in (0, 1):
            o_wait(s)

    return k(x, tab_flat)


def _tc_project(g2, xs, wd1, wl, bl2, wd2, bd2):
    BM = 2048

    def body(g_ref, xs_ref, wd1_ref, wl_ref, bl_ref, wd2_ref, bd_ref, o_ref):
        scal = (
            jnp.dot(xs_ref[...], wl_ref[...], preferred_element_type=jnp.float32)
            + bl_ref[...]
        )
        acc = jnp.dot(g_ref[...], wd1_ref[...], preferred_element_type=jnp.float32)
        acc = acc + jnp.dot(scal, wd2_ref[...], preferred_element_type=jnp.float32)
        o_ref[...] = acc + bd_ref[...]

    d1 = N_CAT * HIDDEN
    return pl.pallas_call(
        body,
        grid=(BATCH // BM,),
        in_specs=[
            pl.BlockSpec((BM, d1), lambda i: (i, 0)),
            pl.BlockSpec((BM, N_SCAL), lambda i: (i, 0)),
            pl.BlockSpec((d1, HIDDEN), lambda i: (0, 0)),
            pl.BlockSpec((N_SCAL, HIDDEN), lambda i: (0, 0)),
            pl.BlockSpec((1, HIDDEN), lambda i: (0, 0)),
            pl.BlockSpec((HIDDEN, HIDDEN), lambda i: (0, 0)),
            pl.BlockSpec((1, HIDDEN), lambda i: (0, 0)),
        ],
        out_specs=pl.BlockSpec((BM, HIDDEN), lambda i: (i, 0)),
        out_shape=jax.ShapeDtypeStruct((BATCH, HIDDEN), jnp.float32),
    )(g2, xs, wd1, wl, bl2, wd2, bd2)


def kernel(x, tables, Wl, bl, Wd, bd):
    tab_flat = tables.reshape(N_CAT * VOCAB, HIDDEN)
    gathered = _sc_gather(x.reshape(-1), tab_flat)
    g2 = gathered.reshape(BATCH, N_CAT * HIDDEN)
    xs = x[:, N_CAT:]
    wd1 = Wd[: N_CAT * HIDDEN]
    wd2 = Wd[N_CAT * HIDDEN :]
    return _tc_project(
        g2, xs, wd1, Wl, bl.reshape(1, HIDDEN), wd2, bd.reshape(1, HIDDEN)
    )
